# Initial kernel scaffold; baseline (speedup 1.0000x reference)
import jax
import jax.numpy as jnp
from jax.experimental import pallas as pl


def kernel(feat, edge_index, W_self0, W_neigh0, b0, W_self1, W_neigh1, b1, W_self2, W_neigh2, b2, W_self3, W_neigh3, b3):
    raise NotImplementedError("write your pallas kernel here")


# speedup vs baseline: 7.7900x; 7.7900x over previous
import jax
import jax.numpy as jnp
from jax import lax
from jax.experimental import pallas as pl
from jax.experimental.pallas import tpu as pltpu
from jax.experimental.pallas import tpu_sc as plsc

N = 10000
E = 320000
D = 128
NC = 2
NS = 16
NT = NC * NS
EPT = E // NT
W = 125
NWIN = EPT // W
QW = 16
NQ = NWIN // QW

_mesh = plsc.VectorSubcoreMesh(
    core_axis_name="c", subcore_axis_name="s", num_cores=NC, num_subcores=NS
)


@jax.jit
def _agg(table, src3, dst3, z128):
  @pl.kernel(
      out_type=jax.ShapeDtypeStruct((NC, N, D), jnp.float32),
      mesh=_mesh,
      scratch_types=(
          pltpu.VMEM_SHARED((N, D), jnp.float32),
          pltpu.VMEM((QW, W), jnp.int32),
          pltpu.VMEM((QW, W), jnp.int32),
          pltpu.VMEM((W, D), jnp.float32),
      ),
  )
  def body(table_h, sidx_h, didx_h, z128_h, part_h, acc_sh, sidx_v, didx_v,
           rows_v):
    c = lax.axis_index("c")
    s = lax.axis_index("s")
    t = c * NS + s

    @pl.when(s == 0)
    def _():
      pltpu.sync_copy(z128_h, acc_sh)

    plsc.subcore_barrier()

    @pl.loop(0, NQ)
    def _(q):
      qb = pl.multiple_of(q * QW, 8)
      pltpu.sync_copy(sidx_h.at[t, pl.ds(qb, QW)], sidx_v)
      pltpu.sync_copy(didx_h.at[t, pl.ds(qb, QW)], didx_v)

      @pl.loop(0, QW)
      def _(w):
        pltpu.sync_copy(table_h.at[sidx_v.at[w]], rows_v)
        pltpu.sync_copy(rows_v, acc_sh.at[didx_v.at[w]], add=True)

    plsc.subcore_barrier()

    @pl.when(s == 0)
    def _():
      pltpu.sync_copy(acc_sh, part_h.at[c])

  return body(table, src3, dst3, z128)


@jax.jit
def _deg(dst3, z128, ones128):
  @pl.kernel(
      out_type=jax.ShapeDtypeStruct((NC, N, D), jnp.float32),
      mesh=_mesh,
      scratch_types=(
          pltpu.VMEM_SHARED((N, D), jnp.float32),
          pltpu.VMEM((QW, W), jnp.int32),
          pltpu.VMEM((W, D), jnp.float32),
      ),
  )
  def body(didx_h, z128_h, ones_h, degpart_h, deg_sh, didx_v, ones_v):
    c = lax.axis_index("c")
    s = lax.axis_index("s")
    t = c * NS + s

    @pl.when(s == 0)
    def _():
      pltpu.sync_copy(z128_h, deg_sh)

    pltpu.sync_copy(ones_h, ones_v)
    plsc.subcore_barrier()

    @pl.loop(0, NQ)
    def _(q):
      qb = pl.multiple_of(q * QW, 8)
      pltpu.sync_copy(didx_h.at[t, pl.ds(qb, QW)], didx_v)

      @pl.loop(0, QW)
      def _(w):
        pltpu.sync_copy(ones_v, deg_sh.at[didx_v.at[w]], add=True)

    plsc.subcore_barrier()

    @pl.when(s == 0)
    def _():
      pltpu.sync_copy(deg_sh, degpart_h.at[c])

  return body(dst3, z128, ones128)


BR = 1000


def _make_layer(relu: bool):
  def body(h_ref, a0_ref, a1_ref, d0_ref, d1_ref, ws_ref, wn_ref, b_ref, o_ref):
    deg = jnp.maximum(d0_ref[0, :, 0:1] + d1_ref[0, :, 0:1], 1.0)
    hn = (a0_ref[0] + a1_ref[0]) / deg
    dn = (((1,), (1,)), ((), ()))
    out = lax.dot_general(h_ref[...], ws_ref[...], dn,
                          preferred_element_type=jnp.float32)
    out = out + lax.dot_general(hn, wn_ref[...], dn,
                                preferred_element_type=jnp.float32)
    out = out + b_ref[...]
    if relu:
      out = jnp.maximum(out, 0.0)
    o_ref[...] = out

  @jax.jit
  def layer(h, part, degpart, w_self, w_neigh, b):
    return pl.pallas_call(
        body,
        out_shape=jax.ShapeDtypeStruct((N, D), jnp.float32),
        grid=(N // BR,),
        in_specs=[
            pl.BlockSpec((BR, D), lambda i: (i, 0)),
            pl.BlockSpec((1, BR, D), lambda i: (0, i, 0)),
            pl.BlockSpec((1, BR, D), lambda i: (1, i, 0)),
            pl.BlockSpec((1, BR, D), lambda i: (0, i, 0)),
            pl.BlockSpec((1, BR, D), lambda i: (1, i, 0)),
            pl.BlockSpec((D, D), lambda i: (0, 0)),
            pl.BlockSpec((D, D), lambda i: (0, 0)),
            pl.BlockSpec((1, D), lambda i: (0, 0)),
        ],
        out_specs=pl.BlockSpec((BR, D), lambda i: (i, 0)),
    )(h, part, part, degpart, degpart, w_self, w_neigh, b)

  return layer


_layer_relu = _make_layer(True)
_layer_last = _make_layer(False)


def kernel(feat, edge_index,
           W_self0, W_neigh0, b0,
           W_self1, W_neigh1, b1,
           W_self2, W_neigh2, b2,
           W_self3, W_neigh3, b3):
  src3 = edge_index[0].astype(jnp.int32).reshape(NT, NWIN, W)
  dst3 = edge_index[1].astype(jnp.int32).reshape(NT, NWIN, W)
  z128 = jnp.zeros((N, D), jnp.float32)

  degpart = _deg(dst3, z128, jnp.ones((W, D), jnp.float32))
  part = _agg(feat, src3, dst3, z128)
  h = _layer_relu(feat, part, degpart, W_self0, W_neigh0, b0.reshape(1, D))
  part = _agg(h, src3, dst3, z128)
  h = _layer_relu(h, part, degpart, W_self1, W_neigh1, b1.reshape(1, D))
  part = _agg(h, src3, dst3, z128)
  h = _layer_relu(h, part, degpart, W_self2, W_neigh2, b2.reshape(1, D))
  part = _agg(h, src3, dst3, z128)
  h = _layer_last(h, part, degpart, W_self3, W_neigh3, b3.reshape(1, D))
  return h


# speedup vs baseline: 9.8993x; 1.2708x over previous
import jax
import jax.numpy as jnp
from jax import lax
from jax.experimental import pallas as pl
from jax.experimental.pallas import tpu as pltpu
from jax.experimental.pallas import tpu_sc as plsc

N = 10000
E = 320000
D = 128
NC = 2
NS = 16
NT = NC * NS
EPT = E // NT
W = 100
NWIN = EPT // W
GW = 10
NQ = NWIN // GW

_mesh = plsc.VectorSubcoreMesh(
    core_axis_name="c", subcore_axis_name="s", num_cores=NC, num_subcores=NS
)


@jax.jit
def _agg(table, src3, dst3, z128):
  @pl.kernel(
      out_type=jax.ShapeDtypeStruct((NC, N, D), jnp.float32),
      mesh=_mesh,
      scratch_types=(
          pltpu.VMEM_SHARED((N, D), jnp.float32),
          pltpu.VMEM((GW, W), jnp.int32),
          pltpu.VMEM((GW, W), jnp.int32),
          pltpu.VMEM((W, D), jnp.float32),
          pltpu.VMEM((W, D), jnp.float32),
          pltpu.SemaphoreType.DMA,
          pltpu.SemaphoreType.DMA,
      ),
  )
  def body(table_h, sidx_h, didx_h, z128_h, part_h, acc_sh, sidx_v, didx_v,
           rows_a, rows_b, sem_a, sem_b):
    c = lax.axis_index("c")
    s = lax.axis_index("s")
    t = c * NS + s

    @pl.when(s == 0)
    def _():
      pltpu.sync_copy(z128_h, acc_sh)

    plsc.subcore_barrier()

    def start(w, buf, sem):
      pltpu.async_copy(table_h.at[sidx_v.at[w]], buf, sem)

    def finish(w, buf, sem):
      pltpu.make_async_copy(table_h.at[sidx_v.at[w]], buf, sem).wait()
      pltpu.sync_copy(buf, acc_sh.at[didx_v.at[w]], add=True)

    @pl.loop(0, NQ)
    def _(q):
      pltpu.sync_copy(sidx_h.at[t, q], sidx_v)
      pltpu.sync_copy(didx_h.at[t, q], didx_v)
      start(0, rows_a, sem_a)

      @pl.loop(0, GW // 2)
      def _(k):
        w = 2 * k
        start(w + 1, rows_b, sem_b)
        finish(w, rows_a, sem_a)

        @pl.when(w + 2 < GW)
        def _():
          start(w + 2, rows_a, sem_a)

        finish(w + 1, rows_b, sem_b)

    plsc.subcore_barrier()

    @pl.when(s == 0)
    def _():
      pltpu.sync_copy(acc_sh, part_h.at[c])

  return body(table, src3, dst3, z128)


@jax.jit
def _deg(dst3, z128, ones128):
  @pl.kernel(
      out_type=jax.ShapeDtypeStruct((NC, N, D), jnp.float32),
      mesh=_mesh,
      scratch_types=(
          pltpu.VMEM_SHARED((N, D), jnp.float32),
          pltpu.VMEM((GW, W), jnp.int32),
          pltpu.VMEM((W, D), jnp.float32),
      ),
  )
  def body(didx_h, z128_h, ones_h, degpart_h, deg_sh, didx_v, ones_v):
    c = lax.axis_index("c")
    s = lax.axis_index("s")
    t = c * NS + s

    @pl.when(s == 0)
    def _():
      pltpu.sync_copy(z128_h, deg_sh)

    pltpu.sync_copy(ones_h, ones_v)
    plsc.subcore_barrier()

    @pl.loop(0, NQ)
    def _(q):
      pltpu.sync_copy(didx_h.at[t, q], didx_v)

      @pl.loop(0, GW)
      def _(w):
        pltpu.sync_copy(ones_v, deg_sh.at[didx_v.at[w]], add=True)

    plsc.subcore_barrier()

    @pl.when(s == 0)
    def _():
      pltpu.sync_copy(deg_sh, degpart_h.at[c])

  return body(dst3, z128, ones128)


BR = 1000


def _make_layer(relu: bool):
  def body(h_ref, a0_ref, a1_ref, d0_ref, d1_ref, ws_ref, wn_ref, b_ref, o_ref):
    deg = jnp.maximum(d0_ref[0, :, 0:1] + d1_ref[0, :, 0:1], 1.0)
    hn = (a0_ref[0] + a1_ref[0]) / deg
    dn = (((1,), (1,)), ((), ()))
    out = lax.dot_general(h_ref[...], ws_ref[...], dn,
                          preferred_element_type=jnp.float32)
    out = out + lax.dot_general(hn, wn_ref[...], dn,
                                preferred_element_type=jnp.float32)
    out = out + b_ref[...]
    if relu:
      out = jnp.maximum(out, 0.0)
    o_ref[...] = out

  @jax.jit
  def layer(h, part, degpart, w_self, w_neigh, b):
    return pl.pallas_call(
        body,
        out_shape=jax.ShapeDtypeStruct((N, D), jnp.float32),
        grid=(N // BR,),
        in_specs=[
            pl.BlockSpec((BR, D), lambda i: (i, 0)),
            pl.BlockSpec((1, BR, D), lambda i: (0, i, 0)),
            pl.BlockSpec((1, BR, D), lambda i: (1, i, 0)),
            pl.BlockSpec((1, BR, D), lambda i: (0, i, 0)),
            pl.BlockSpec((1, BR, D), lambda i: (1, i, 0)),
            pl.BlockSpec((D, D), lambda i: (0, 0)),
            pl.BlockSpec((D, D), lambda i: (0, 0)),
            pl.BlockSpec((1, D), lambda i: (0, 0)),
        ],
        out_specs=pl.BlockSpec((BR, D), lambda i: (i, 0)),
    )(h, part, part, degpart, degpart, w_self, w_neigh, b)

  return layer


_layer_relu = _make_layer(True)
_layer_last = _make_layer(False)


def kernel(feat, edge_index,
           W_self0, W_neigh0, b0,
           W_self1, W_neigh1, b1,
           W_self2, W_neigh2, b2,
           W_self3, W_neigh3, b3):
  src3 = edge_index[0].astype(jnp.int32).reshape(NT, NQ, GW, W)
  dst3 = edge_index[1].astype(jnp.int32).reshape(NT, NQ, GW, W)
  z128 = jnp.zeros((N, D), jnp.float32)

  degpart = _deg(dst3, z128, jnp.ones((W, D), jnp.float32))
  part = _agg(feat, src3, dst3, z128)
  h = _layer_relu(feat, part, degpart, W_self0, W_neigh0, b0.reshape(1, D))
  part = _agg(h, src3, dst3, z128)
  h = _layer_relu(h, part, degpart, W_self1, W_neigh1, b1.reshape(1, D))
  part = _agg(h, src3, dst3, z128)
  h = _layer_relu(h, part, degpart, W_self2, W_neigh2, b2.reshape(1, D))
  part = _agg(h, src3, dst3, z128)
  h = _layer_last(h, part, degpart, W_self3, W_neigh3, b3.reshape(1, D))
  return h


# speedup vs baseline: 10.0509x; 1.0153x over previous
import jax
import jax.numpy as jnp
from jax import lax
from jax.experimental import pallas as pl
from jax.experimental.pallas import tpu as pltpu
from jax.experimental.pallas import tpu_sc as plsc

N = 10000
E = 320000
D = 128
NC = 2
NS = 16
NT = NC * NS
EPT = E // NT
W = 125
NWIN = EPT // W
GW = 8
NQ = NWIN // GW

_mesh = plsc.VectorSubcoreMesh(
    core_axis_name="c", subcore_axis_name="s", num_cores=NC, num_subcores=NS
)


@jax.jit
def _agg(table, src3, dst3, z128):
  @pl.kernel(
      out_type=jax.ShapeDtypeStruct((NC, N, D), jnp.float32),
      mesh=_mesh,
      scratch_types=(
          pltpu.VMEM_SHARED((N, D), jnp.float32),
          pltpu.VMEM((GW, W), jnp.int32),
          pltpu.VMEM((GW, W), jnp.int32),
          pltpu.VMEM((W, D), jnp.float32),
          pltpu.VMEM((W, D), jnp.float32),
          pltpu.SemaphoreType.DMA,
          pltpu.SemaphoreType.DMA,
      ),
  )
  def body(table_h, sidx_h, didx_h, z128_h, part_h, acc_sh, sidx_v, didx_v,
           rows_a, rows_b, sem_a, sem_b):
    c = lax.axis_index("c")
    s = lax.axis_index("s")
    t = c * NS + s

    @pl.when(s == 0)
    def _():
      pltpu.sync_copy(z128_h, acc_sh)

    plsc.subcore_barrier()

    def start(w, buf, sem):
      pltpu.async_copy(table_h.at[sidx_v.at[w]], buf, sem)

    def finish(w, buf, sem):
      pltpu.make_async_copy(table_h.at[sidx_v.at[w]], buf, sem).wait()
      pltpu.sync_copy(buf, acc_sh.at[didx_v.at[w]], add=True)

    @pl.loop(0, NQ)
    def _(q):
      pltpu.sync_copy(sidx_h.at[t, q], sidx_v)
      pltpu.sync_copy(didx_h.at[t, q], didx_v)
      start(0, rows_a, sem_a)

      @pl.loop(0, GW // 2)
      def _(k):
        w = 2 * k
        start(w + 1, rows_b, sem_b)
        finish(w, rows_a, sem_a)

        @pl.when(w + 2 < GW)
        def _():
          start(w + 2, rows_a, sem_a)

        finish(w + 1, rows_b, sem_b)

    plsc.subcore_barrier()

    @pl.when(s == 0)
    def _():
      pltpu.sync_copy(acc_sh, part_h.at[c])

  return body(table, src3, dst3, z128)


@jax.jit
def _deg(dst3, z128, ones128):
  @pl.kernel(
      out_type=jax.ShapeDtypeStruct((NC, N, D), jnp.float32),
      mesh=_mesh,
      scratch_types=(
          pltpu.VMEM_SHARED((N, D), jnp.float32),
          pltpu.VMEM((GW, W), jnp.int32),
          pltpu.VMEM((W, D), jnp.float32),
      ),
  )
  def body(didx_h, z128_h, ones_h, degpart_h, deg_sh, didx_v, ones_v):
    c = lax.axis_index("c")
    s = lax.axis_index("s")
    t = c * NS + s

    @pl.when(s == 0)
    def _():
      pltpu.sync_copy(z128_h, deg_sh)

    pltpu.sync_copy(ones_h, ones_v)
    plsc.subcore_barrier()

    @pl.loop(0, NQ)
    def _(q):
      pltpu.sync_copy(didx_h.at[t, q], didx_v)

      @pl.loop(0, GW)
      def _(w):
        pltpu.sync_copy(ones_v, deg_sh.at[didx_v.at[w]], add=True)

    plsc.subcore_barrier()

    @pl.when(s == 0)
    def _():
      pltpu.sync_copy(deg_sh, degpart_h.at[c])

  return body(dst3, z128, ones128)


BR = 1000


def _make_layer(relu: bool):
  def body(h_ref, a0_ref, a1_ref, d0_ref, d1_ref, ws_ref, wn_ref, b_ref, o_ref):
    deg = jnp.maximum(d0_ref[0, :, 0:1] + d1_ref[0, :, 0:1], 1.0)
    hn = (a0_ref[0] + a1_ref[0]) / deg
    dn = (((1,), (1,)), ((), ()))
    out = lax.dot_general(h_ref[...], ws_ref[...], dn,
                          preferred_element_type=jnp.float32)
    out = out + lax.dot_general(hn, wn_ref[...], dn,
                                preferred_element_type=jnp.float32)
    out = out + b_ref[...]
    if relu:
      out = jnp.maximum(out, 0.0)
    o_ref[...] = out

  @jax.jit
  def layer(h, part, degpart, w_self, w_neigh, b):
    return pl.pallas_call(
        body,
        out_shape=jax.ShapeDtypeStruct((N, D), jnp.float32),
        grid=(N // BR,),
        in_specs=[
            pl.BlockSpec((BR, D), lambda i: (i, 0)),
            pl.BlockSpec((1, BR, D), lambda i: (0, i, 0)),
            pl.BlockSpec((1, BR, D), lambda i: (1, i, 0)),
            pl.BlockSpec((1, BR, D), lambda i: (0, i, 0)),
            pl.BlockSpec((1, BR, D), lambda i: (1, i, 0)),
            pl.BlockSpec((D, D), lambda i: (0, 0)),
            pl.BlockSpec((D, D), lambda i: (0, 0)),
            pl.BlockSpec((1, D), lambda i: (0, 0)),
        ],
        out_specs=pl.BlockSpec((BR, D), lambda i: (i, 0)),
    )(h, part, part, degpart, degpart, w_self, w_neigh, b)

  return layer


_layer_relu = _make_layer(True)
_layer_last = _make_layer(False)


def kernel(feat, edge_index,
           W_self0, W_neigh0, b0,
           W_self1, W_neigh1, b1,
           W_self2, W_neigh2, b2,
           W_self3, W_neigh3, b3):
  src3 = edge_index[0].astype(jnp.int32).reshape(NT, NQ, GW, W)
  dst3 = edge_index[1].astype(jnp.int32).reshape(NT, NQ, GW, W)
  z128 = jnp.zeros((N, D), jnp.float32)

  degpart = _deg(dst3, z128, jnp.ones((W, D), jnp.float32))
  part = _agg(feat, src3, dst3, z128)
  h = _layer_relu(feat, part, degpart, W_self0, W_neigh0, b0.reshape(1, D))
  part = _agg(h, src3, dst3, z128)
  h = _layer_relu(h, part, degpart, W_self1, W_neigh1, b1.reshape(1, D))
  part = _agg(h, src3, dst3, z128)
  h = _layer_relu(h, part, degpart, W_self2, W_neigh2, b2.reshape(1, D))
  part = _agg(h, src3, dst3, z128)
  h = _layer_last(h, part, degpart, W_self3, W_neigh3, b3.reshape(1, D))
  return h


# speedup vs baseline: 10.9077x; 1.0852x over previous
import jax
import jax.numpy as jnp
from jax import lax
from jax.experimental import pallas as pl
from jax.experimental.pallas import tpu as pltpu
from jax.experimental.pallas import tpu_sc as plsc

N = 10000
E = 320000
D = 128
NC = 2
NS = 16
NT = NC * NS
EPT = E // NT
W = 125
NWIN = EPT // W
GW = 8
NQ = NWIN // GW

_mesh = plsc.VectorSubcoreMesh(
    core_axis_name="c", subcore_axis_name="s", num_cores=NC, num_subcores=NS
)


@jax.jit
def _agg(table, src3, dst3, z128):
  @pl.kernel(
      out_type=jax.ShapeDtypeStruct((NC, N, D), jnp.float32),
      mesh=_mesh,
      scratch_types=(
          pltpu.VMEM_SHARED((N, D), jnp.float32),
          pltpu.VMEM((GW, W), jnp.int32),
          pltpu.VMEM((GW, W), jnp.int32),
          pltpu.VMEM((W, D), jnp.float32),
          pltpu.VMEM((W, D), jnp.float32),
          pltpu.SemaphoreType.DMA,
          pltpu.SemaphoreType.DMA,
      ),
  )
  def body(table_h, sidx_h, didx_h, z128_h, part_h, acc_sh, sidx_v, didx_v,
           rows_a, rows_b, sem_a, sem_b):
    c = lax.axis_index("c")
    s = lax.axis_index("s")
    t = c * NS + s

    @pl.when(s == 0)
    def _():
      pltpu.sync_copy(z128_h, acc_sh)

    plsc.subcore_barrier()

    def start(w, buf, sem):
      pltpu.async_copy(table_h.at[sidx_v.at[w]], buf, sem)

    def finish(w, buf, sem):
      pltpu.make_async_copy(table_h.at[sidx_v.at[w]], buf, sem).wait()
      pltpu.sync_copy(buf, acc_sh.at[didx_v.at[w]], add=True)

    @pl.loop(0, NQ)
    def _(q):
      pltpu.sync_copy(sidx_h.at[t, q], sidx_v)
      pltpu.sync_copy(didx_h.at[t, q], didx_v)
      start(0, rows_a, sem_a)

      @pl.loop(0, GW // 2)
      def _(k):
        w = 2 * k
        start(w + 1, rows_b, sem_b)
        finish(w, rows_a, sem_a)

        @pl.when(w + 2 < GW)
        def _():
          start(w + 2, rows_a, sem_a)

        finish(w + 1, rows_b, sem_b)

    plsc.subcore_barrier()

    @pl.when(s == 0)
    def _():
      pltpu.sync_copy(acc_sh, part_h.at[c])

  return body(table, src3, dst3, z128)


DW = 16


@jax.jit
def _deg(dst3, z16, ones16):
  @pl.kernel(
      out_type=jax.ShapeDtypeStruct((NC, N, DW), jnp.float32),
      mesh=_mesh,
      compiler_params=pltpu.CompilerParams(use_tc_tiling_on_sc=False),
      scratch_types=(
          pltpu.VMEM_SHARED((N, DW), jnp.float32),
          pltpu.VMEM((GW, W), jnp.int32),
          pltpu.VMEM((W, DW), jnp.float32),
      ),
  )
  def body(didx_h, z16_h, ones_h, degpart_h, deg_sh, didx_v, ones_v):
    c = lax.axis_index("c")
    s = lax.axis_index("s")
    t = c * NS + s

    @pl.when(s == 0)
    def _():
      pltpu.sync_copy(z16_h, deg_sh)

    pltpu.sync_copy(ones_h, ones_v)
    plsc.subcore_barrier()

    @pl.loop(0, NQ)
    def _(q):
      pltpu.sync_copy(didx_h.at[t, q], didx_v)

      @pl.loop(0, GW)
      def _(w):
        pltpu.sync_copy(ones_v, deg_sh.at[didx_v.at[w]], add=True)

    plsc.subcore_barrier()

    @pl.when(s == 0)
    def _():
      pltpu.sync_copy(deg_sh, degpart_h.at[c])

  return body(dst3, z16, ones16)


BR = 1000


def _make_layer(relu: bool):
  def body(h_ref, a0_ref, a1_ref, d0_ref, d1_ref, ws_ref, wn_ref, b_ref, o_ref):
    deg = jnp.maximum(d0_ref[0, :, 0:1] + d1_ref[0, :, 0:1], 1.0)
    hn = (a0_ref[0] + a1_ref[0]) / deg
    dn = (((1,), (1,)), ((), ()))
    out = lax.dot_general(h_ref[...], ws_ref[...], dn,
                          preferred_element_type=jnp.float32)
    out = out + lax.dot_general(hn, wn_ref[...], dn,
                                preferred_element_type=jnp.float32)
    out = out + b_ref[...]
    if relu:
      out = jnp.maximum(out, 0.0)
    o_ref[...] = out

  @jax.jit
  def layer(h, part, degpart, w_self, w_neigh, b):
    return pl.pallas_call(
        body,
        out_shape=jax.ShapeDtypeStruct((N, D), jnp.float32),
        grid=(N // BR,),
        in_specs=[
            pl.BlockSpec((BR, D), lambda i: (i, 0)),
            pl.BlockSpec((1, BR, D), lambda i: (0, i, 0)),
            pl.BlockSpec((1, BR, D), lambda i: (1, i, 0)),
            pl.BlockSpec((1, BR, DW), lambda i: (0, i, 0)),
            pl.BlockSpec((1, BR, DW), lambda i: (1, i, 0)),
            pl.BlockSpec((D, D), lambda i: (0, 0)),
            pl.BlockSpec((D, D), lambda i: (0, 0)),
            pl.BlockSpec((1, D), lambda i: (0, 0)),
        ],
        out_specs=pl.BlockSpec((BR, D), lambda i: (i, 0)),
    )(h, part, part, degpart, degpart, w_self, w_neigh, b)

  return layer


_layer_relu = _make_layer(True)
_layer_last = _make_layer(False)


def kernel(feat, edge_index,
           W_self0, W_neigh0, b0,
           W_self1, W_neigh1, b1,
           W_self2, W_neigh2, b2,
           W_self3, W_neigh3, b3):
  src3 = edge_index[0].astype(jnp.int32).reshape(NT, NQ, GW, W)
  dst3 = edge_index[1].astype(jnp.int32).reshape(NT, NQ, GW, W)
  z128 = jnp.zeros((N, D), jnp.float32)

  degpart = _deg(dst3, jnp.zeros((N, DW), jnp.float32),
                 jnp.ones((W, DW), jnp.float32))
  part = _agg(feat, src3, dst3, z128)
  h = _layer_relu(feat, part, degpart, W_self0, W_neigh0, b0.reshape(1, D))
  part = _agg(h, src3, dst3, z128)
  h = _layer_relu(h, part, degpart, W_self1, W_neigh1, b1.reshape(1, D))
  part = _agg(h, src3, dst3, z128)
  h = _layer_relu(h, part, degpart, W_self2, W_neigh2, b2.reshape(1, D))
  part = _agg(h, src3, dst3, z128)
  h = _layer_last(h, part, degpart, W_self3, W_neigh3, b3.reshape(1, D))
  return h


# speedup vs baseline: 11.7648x; 1.0786x over previous
import jax
import jax.numpy as jnp
from jax import lax
from jax.experimental import pallas as pl
from jax.experimental.pallas import tpu as pltpu
from jax.experimental.pallas import tpu_sc as plsc

N = 10000
E = 320000
D = 128
NC = 2
NS = 16
NT = NC * NS
EPT = E // NT
W = 125
NWIN = EPT // W
GW = 8
NQ = NWIN // GW

_mesh = plsc.VectorSubcoreMesh(
    core_axis_name="c", subcore_axis_name="s", num_cores=NC, num_subcores=NS
)


RPT = N // NS


@jax.jit
def _agg(table, idx5, z128):
  @pl.kernel(
      out_type=jax.ShapeDtypeStruct((NC, N, D), jnp.float32),
      mesh=_mesh,
      compiler_params=pltpu.CompilerParams(use_tc_tiling_on_sc=False),
      scratch_types=(
          pltpu.VMEM_SHARED((N, D), jnp.float32),
          pltpu.VMEM((2, GW, W), jnp.int32),
          pltpu.VMEM((2, GW, W), jnp.int32),
          pltpu.VMEM((W, D), jnp.float32),
          pltpu.VMEM((W, D), jnp.float32),
          pltpu.SemaphoreType.DMA,
          pltpu.SemaphoreType.DMA,
          pltpu.SemaphoreType.DMA,
          pltpu.SemaphoreType.DMA,
          pltpu.SemaphoreType.DMA,
      ),
  )
  def body(table_h, idx_h, z128_h, part_h, acc_sh, idx_a, idx_b,
           rows_a, rows_b, sem_a, sem_b, sem_ia, sem_ib, sem_z):
    c = lax.axis_index("c")
    s = lax.axis_index("s")
    t = c * NS + s
    rb = s * RPT

    pltpu.async_copy(z128_h.at[pl.ds(rb, RPT)], acc_sh.at[pl.ds(rb, RPT)],
                     sem_z)
    pltpu.async_copy(idx_h.at[t, 0], idx_a, sem_ia)
    pltpu.async_copy(idx_h.at[t, 1], idx_b, sem_ib)
    pltpu.make_async_copy(z128_h.at[pl.ds(rb, RPT)],
                          acc_sh.at[pl.ds(rb, RPT)], sem_z).wait()
    plsc.subcore_barrier()

    def start(ib, w, buf, sem):
      pltpu.async_copy(table_h.at[ib.at[0, w]], buf, sem)

    def finish(ib, w, buf, sem):
      pltpu.make_async_copy(table_h.at[ib.at[0, w]], buf, sem).wait()
      pltpu.sync_copy(buf, acc_sh.at[ib.at[1, w]], add=True)

    def process_group(ib):
      start(ib, 0, rows_a, sem_a)

      @pl.loop(0, GW // 2)
      def _(k):
        w = 2 * k
        start(ib, w + 1, rows_b, sem_b)
        finish(ib, w, rows_a, sem_a)

        @pl.when(w + 2 < GW)
        def _():
          start(ib, w + 2, rows_a, sem_a)

        finish(ib, w + 1, rows_b, sem_b)

    @pl.loop(0, NQ, step=2)
    def _(q):
      pltpu.make_async_copy(idx_h.at[t, q], idx_a, sem_ia).wait()
      process_group(idx_a)

      @pl.when(q + 2 < NQ)
      def _():
        pltpu.async_copy(idx_h.at[t, q + 2], idx_a, sem_ia)

      pltpu.make_async_copy(idx_h.at[t, q + 1], idx_b, sem_ib).wait()
      process_group(idx_b)

      @pl.when(q + 3 < NQ)
      def _():
        pltpu.async_copy(idx_h.at[t, q + 3], idx_b, sem_ib)

    plsc.subcore_barrier()
    pltpu.sync_copy(acc_sh.at[pl.ds(rb, RPT)],
                    part_h.at[c, pl.ds(rb, RPT)])

  return body(table, idx5, z128)


DW = 16


@jax.jit
def _deg(dst3, z16, ones16):
  @pl.kernel(
      out_type=jax.ShapeDtypeStruct((NC, N, DW), jnp.float32),
      mesh=_mesh,
      compiler_params=pltpu.CompilerParams(use_tc_tiling_on_sc=False),
      scratch_types=(
          pltpu.VMEM_SHARED((N, DW), jnp.float32),
          pltpu.VMEM((GW, W), jnp.int32),
          pltpu.VMEM((W, DW), jnp.float32),
      ),
  )
  def body(didx_h, z16_h, ones_h, degpart_h, deg_sh, didx_v, ones_v):
    c = lax.axis_index("c")
    s = lax.axis_index("s")
    t = c * NS + s

    @pl.when(s == 0)
    def _():
      pltpu.sync_copy(z16_h, deg_sh)

    pltpu.sync_copy(ones_h, ones_v)
    plsc.subcore_barrier()

    @pl.loop(0, NQ)
    def _(q):
      pltpu.sync_copy(didx_h.at[t, q], didx_v)

      @pl.loop(0, GW)
      def _(w):
        pltpu.sync_copy(ones_v, deg_sh.at[didx_v.at[w]], add=True)

    plsc.subcore_barrier()

    @pl.when(s == 0)
    def _():
      pltpu.sync_copy(deg_sh, degpart_h.at[c])

  return body(dst3, z16, ones16)


BR = 1000


def _make_layer(relu: bool):
  def body(h_ref, a0_ref, a1_ref, d0_ref, d1_ref, ws_ref, wn_ref, b_ref, o_ref):
    deg = jnp.maximum(d0_ref[0, :, 0:1] + d1_ref[0, :, 0:1], 1.0)
    hn = (a0_ref[0] + a1_ref[0]) / deg
    dn = (((1,), (1,)), ((), ()))
    out = lax.dot_general(h_ref[...], ws_ref[...], dn,
                          preferred_element_type=jnp.float32)
    out = out + lax.dot_general(hn, wn_ref[...], dn,
                                preferred_element_type=jnp.float32)
    out = out + b_ref[...]
    if relu:
      out = jnp.maximum(out, 0.0)
    o_ref[...] = out

  @jax.jit
  def layer(h, part, degpart, w_self, w_neigh, b):
    return pl.pallas_call(
        body,
        out_shape=jax.ShapeDtypeStruct((N, D), jnp.float32),
        grid=(N // BR,),
        in_specs=[
            pl.BlockSpec((BR, D), lambda i: (i, 0)),
            pl.BlockSpec((1, BR, D), lambda i: (0, i, 0)),
            pl.BlockSpec((1, BR, D), lambda i: (1, i, 0)),
            pl.BlockSpec((1, BR, DW), lambda i: (0, i, 0)),
            pl.BlockSpec((1, BR, DW), lambda i: (1, i, 0)),
            pl.BlockSpec((D, D), lambda i: (0, 0)),
            pl.BlockSpec((D, D), lambda i: (0, 0)),
            pl.BlockSpec((1, D), lambda i: (0, 0)),
        ],
        out_specs=pl.BlockSpec((BR, D), lambda i: (i, 0)),
    )(h, part, part, degpart, degpart, w_self, w_neigh, b)

  return layer


_layer_relu = _make_layer(True)
_layer_last = _make_layer(False)


def kernel(feat, edge_index,
           W_self0, W_neigh0, b0,
           W_self1, W_neigh1, b1,
           W_self2, W_neigh2, b2,
           W_self3, W_neigh3, b3):
  src3 = edge_index[0].astype(jnp.int32).reshape(NT, NQ, GW, W)
  dst3 = edge_index[1].astype(jnp.int32).reshape(NT, NQ, GW, W)
  idx5 = jnp.stack([src3, dst3], axis=2)
  z128 = jnp.zeros((N, D), jnp.float32)

  degpart = _deg(dst3, jnp.zeros((N, DW), jnp.float32),
                 jnp.ones((W, DW), jnp.float32))
  part = _agg(feat, idx5, z128)
  h = _layer_relu(feat, part, degpart, W_self0, W_neigh0, b0.reshape(1, D))
  part = _agg(h, idx5, z128)
  h = _layer_relu(h, part, degpart, W_self1, W_neigh1, b1.reshape(1, D))
  part = _agg(h, idx5, z128)
  h = _layer_relu(h, part, degpart, W_self2, W_neigh2, b2.reshape(1, D))
  part = _agg(h, idx5, z128)
  h = _layer_last(h, part, degpart, W_self3, W_neigh3, b3.reshape(1, D))
  return h
